# baseline (speedup 1.0000x reference)
import functools

import jax
import jax.numpy as jnp
from jax import lax
from jax.experimental import pallas as pl
from jax.experimental.pallas import tpu as pltpu
from jax.experimental.pallas import tpu_sc as plsc

BATCH = 32
SEQ = 2048
D = 128
NUM_CORES = 2
NUM_SUBCORES = 16
HALF = BATCH // NUM_CORES
SCHUNK = SEQ // NUM_SUBCORES
VECS = D // 16
NSTEP = HALF
NBUF = 4
UNROLL = 4


@functools.partial(
    pl.kernel,
    mesh=plsc.VectorSubcoreMesh(core_axis_name="c", subcore_axis_name="s"),
    out_type=jax.ShapeDtypeStruct((BATCH, SEQ, D), jnp.float32),
    scratch_types=(
        [pltpu.VMEM((HALF, SCHUNK), jnp.int32)]
        + [pltpu.VMEM((SCHUNK, D), jnp.float32)]
        + [pltpu.VMEM((SCHUNK, D), jnp.float32) for _ in range(NBUF)]
        + [pltpu.SemaphoreType.DMA for _ in range(2 * NBUF)]
    ),
)
def _emb_kernel(idx_hbm, table_hbm, pos_hbm, out_hbm, *refs):
    idx_v = refs[0]
    pos_v = refs[1]
    rows_v = refs[2:2 + NBUF]
    gsem = refs[2 + NBUF:2 + 2 * NBUF]
    wsem = refs[2 + 2 * NBUF:2 + 3 * NBUF]

    half = lax.axis_index("c")
    chunk = lax.axis_index("s")
    b0 = half * HALF
    base = chunk * SCHUNK

    def start_gather(g, p):
        pltpu.async_copy(table_hbm.at[idx_v.at[g]], rows_v[p], gsem[p])

    def wait_gather(g, p):
        pltpu.make_async_copy(
            table_hbm.at[idx_v.at[g]], rows_v[p], gsem[p]).wait()

    def start_write(g, p):
        pltpu.async_copy(rows_v[p],
                         out_hbm.at[b0 + g, pl.ds(base, SCHUNK), :], wsem[p])

    def wait_write(p):
        pltpu.make_async_copy(
            rows_v[p], out_hbm.at[0, pl.ds(0, SCHUNK), :], wsem[p]).wait()

    def add_pos2(pa, pb):
        ra, rb = rows_v[pa], rows_v[pb]

        def add_body(r2, carry):
            for u in range(2):
                r = r2 * 2 + u
                for cc in range(VECS):
                    sl = pl.ds(cc * 16, 16)
                    pv = pos_v[r, sl]
                    plsc.addupdate(ra.at[r, sl], pv)
                    plsc.addupdate(rb.at[r, sl], pv)
            return carry

        lax.fori_loop(0, SCHUNK // 2, add_body, 0)

    pltpu.sync_copy(pos_hbm.at[0, pl.ds(base, SCHUNK), :], pos_v)
    pltpu.sync_copy(idx_hbm.at[pl.ds(b0, HALF), pl.ds(base, SCHUNK)], idx_v)
    start_gather(0, 0)

    def group_body(h, carry):
        g0 = h * UNROLL
        for pair in range(2):
            pa, pb = 2 * pair, 2 * pair + 1
            @pl.when(h >= 1)
            def _():
                wait_write(pb)
            start_gather(g0 + pa + 1, pb)
            wait_gather(g0 + pa, pa)

            nb = (pb + 1) % NBUF
            if pair == 0:
                @pl.when(h >= 1)
                def _():
                    wait_write(nb)
                start_gather(g0 + pb + 1, nb)
            else:
                @pl.when(h < NSTEP // UNROLL - 1)
                def _():
                    wait_write(nb)
                    start_gather(g0 + pb + 1, nb)
            wait_gather(g0 + pb, pb)
            add_pos2(pa, pb)
            start_write(g0 + pa, pa)
            start_write(g0 + pb, pb)
        return carry

    lax.fori_loop(0, NSTEP // UNROLL, group_body, 0)

    for p in range(NBUF):
        wait_write(p)


def kernel(input_ids, emb_table, pos_enc):
    return _emb_kernel(input_ids.astype(jnp.int32), emb_table, pos_enc)


# speedup vs baseline: 1.1180x; 1.1180x over previous
import functools

import jax
import jax.numpy as jnp
from jax import lax
from jax.experimental import pallas as pl
from jax.experimental.pallas import tpu as pltpu
from jax.experimental.pallas import tpu_sc as plsc

BATCH = 32
SEQ = 2048
D = 128
NUM_CORES = 2
NUM_SUBCORES = 16
HALF = BATCH // NUM_CORES
SCHUNK = SEQ // NUM_SUBCORES
VECS = D // 16
NSTEP = HALF
NBUF = 4
UNROLL = 4


@functools.partial(
    pl.kernel,
    mesh=plsc.VectorSubcoreMesh(core_axis_name="c", subcore_axis_name="s"),
    out_type=jax.ShapeDtypeStruct((BATCH, SEQ, D), jnp.float32),
    scratch_types=(
        [pltpu.VMEM((HALF, SCHUNK), jnp.int32)]
        + [pltpu.VMEM((SCHUNK, D), jnp.float32)]
        + [pltpu.VMEM((SCHUNK, D), jnp.float32) for _ in range(NBUF)]
        + [pltpu.SemaphoreType.DMA for _ in range(2 * NBUF)]
    ),
)
def _emb_kernel(idx_hbm, table_hbm, pos_hbm, out_hbm, *refs):
    idx_v = refs[0]
    pos_v = refs[1]
    rows_v = refs[2:2 + NBUF]
    gsem = refs[2 + NBUF:2 + 2 * NBUF]
    wsem = refs[2 + 2 * NBUF:2 + 3 * NBUF]

    half = lax.axis_index("c")
    chunk = lax.axis_index("s")
    b0 = half * HALF
    base = chunk * SCHUNK

    def start_gather(g, p):
        pltpu.async_copy(table_hbm.at[idx_v.at[g]], rows_v[p], gsem[p])

    def wait_gather(g, p):
        pltpu.make_async_copy(
            table_hbm.at[idx_v.at[g]], rows_v[p], gsem[p]).wait()

    def start_write(g, p):
        pltpu.async_copy(rows_v[p],
                         out_hbm.at[b0 + g, pl.ds(base, SCHUNK), :], wsem[p])

    def wait_write(p):
        pltpu.make_async_copy(
            rows_v[p], out_hbm.at[0, pl.ds(0, SCHUNK), :], wsem[p]).wait()

    def add_pos(p):
        rows = rows_v[p]

        def add_body(r2, carry):
            for u in range(2):
                r = r2 * 2 + u
                for cc in range(VECS):
                    sl = pl.ds(cc * 16, 16)
                    plsc.addupdate(rows.at[r, sl], pos_v[r, sl])
            return carry

        lax.fori_loop(0, SCHUNK // 2, add_body, 0)

    pltpu.sync_copy(pos_hbm.at[0, pl.ds(base, SCHUNK), :], pos_v)
    pltpu.sync_copy(idx_hbm.at[pl.ds(b0, HALF), pl.ds(base, SCHUNK)], idx_v)
    start_gather(0, 0)
    start_gather(1, 1)

    def step(g, p, h):
        q = (p + 2) % NBUF

        def launch_ahead():
            if p <= 1:
                @pl.when(h >= 1)
                def _():
                    wait_write(q)
            else:
                wait_write(q)
            start_gather(g + 2, q)

        if p <= 1:
            launch_ahead()
        else:
            @pl.when(h < NSTEP // UNROLL - 1)
            def _():
                launch_ahead()

        wait_gather(g, p)
        add_pos(p)
        start_write(g, p)

    def group_body(h, carry):
        for p in range(UNROLL):
            step(h * UNROLL + p, p, h)
        return carry

    lax.fori_loop(0, NSTEP // UNROLL, group_body, 0)

    for p in range(NBUF):
        wait_write(p)


def kernel(input_ids, emb_table, pos_enc):
    return _emb_kernel(input_ids.astype(jnp.int32), emb_table, pos_enc)


# speedup vs baseline: 1.1442x; 1.0234x over previous
import functools

import jax
import jax.numpy as jnp
from jax import lax
from jax.experimental import pallas as pl
from jax.experimental.pallas import tpu as pltpu
from jax.experimental.pallas import tpu_sc as plsc

BATCH = 32
SEQ = 2048
D = 128
NUM_CORES = 2
NUM_SUBCORES = 16
HALF = BATCH // NUM_CORES
SCHUNK = SEQ // NUM_SUBCORES
VECS = D // 16
NSTEP = HALF
NBUF = 4
UNROLL = 4


@functools.partial(
    pl.kernel,
    mesh=plsc.VectorSubcoreMesh(core_axis_name="c", subcore_axis_name="s"),
    out_type=jax.ShapeDtypeStruct((BATCH, SEQ, D), jnp.float32),
    scratch_types=(
        [pltpu.VMEM((HALF, SCHUNK), jnp.int32)]
        + [pltpu.VMEM((SCHUNK, D), jnp.float32)]
        + [pltpu.VMEM((SCHUNK, D), jnp.float32) for _ in range(NBUF)]
        + [pltpu.SemaphoreType.DMA for _ in range(2 * NBUF)]
    ),
)
def _emb_kernel(idx_hbm, table_hbm, pos_hbm, out_hbm, *refs):
    idx_v = refs[0]
    pos_v = refs[1]
    rows_v = refs[2:2 + NBUF]
    gsem = refs[2 + NBUF:2 + 2 * NBUF]
    wsem = refs[2 + 2 * NBUF:2 + 3 * NBUF]

    half = lax.axis_index("c")
    chunk = lax.axis_index("s")
    b0 = half * HALF
    base = chunk * SCHUNK

    def start_gather(g, p):
        pltpu.async_copy(table_hbm.at[idx_v.at[g]], rows_v[p], gsem[p])

    def wait_gather(g, p):
        pltpu.make_async_copy(
            table_hbm.at[idx_v.at[g]], rows_v[p], gsem[p]).wait()

    def start_write(g, p):
        pltpu.async_copy(rows_v[p],
                         out_hbm.at[b0 + g, pl.ds(base, SCHUNK), :], wsem[p])

    def wait_write(p):
        pltpu.make_async_copy(
            rows_v[p], out_hbm.at[0, pl.ds(0, SCHUNK), :], wsem[p]).wait()

    def add_pos(p):
        rows = rows_v[p]

        def add_body(r2, carry):
            for u in range(2):
                r = r2 * 2 + u
                for cc in range(VECS):
                    sl = pl.ds(cc * 16, 16)
                    plsc.addupdate(rows.at[r, sl], pos_v[r, sl])
            return carry

        lax.fori_loop(0, SCHUNK // 2, add_body, 0)

    pltpu.sync_copy(idx_hbm.at[pl.ds(b0, HALF), pl.ds(base, SCHUNK)], idx_v)
    start_gather(0, 0)
    start_gather(1, 1)
    pltpu.sync_copy(pos_hbm.at[0, pl.ds(base, SCHUNK), :], pos_v)

    def step(g, p, h):
        q = (p + 2) % NBUF

        def launch_ahead():
            if p <= 1:
                @pl.when(h >= 1)
                def _():
                    wait_write(q)
            else:
                wait_write(q)
            start_gather(g + 2, q)

        if p <= 1:
            launch_ahead()
        else:
            @pl.when(h < NSTEP // UNROLL - 1)
            def _():
                launch_ahead()

        wait_gather(g, p)
        add_pos(p)
        start_write(g, p)

    def group_body(h, carry):
        for p in range(UNROLL):
            step(h * UNROLL + p, p, h)
        return carry

    lax.fori_loop(0, NSTEP // UNROLL, group_body, 0)

    for p in range(NBUF):
        wait_write(p)


def kernel(input_ids, emb_table, pos_enc):
    return _emb_kernel(input_ids.astype(jnp.int32), emb_table, pos_enc)
